# baseline (speedup 1.0000x reference)
import functools

import jax
import jax.numpy as jnp
from jax.experimental import pallas as pl
from jax.experimental.pallas import tpu as pltpu

_BR_P = 200
_BR = 400


def _prep_body(br, adj_ref, x_ref, w0_ref, abf_ref, dis_ref, z1_ref):
    i = pl.program_id(0)
    blk = adj_ref[...]
    rows = jax.lax.broadcasted_iota(jnp.int32, blk.shape, 0) + i * br
    cols = jax.lax.broadcasted_iota(jnp.int32, blk.shape, 1)
    blk = jnp.where(rows == cols, 1.0, blk)
    abf_ref[...] = blk.astype(jnp.bfloat16)
    dis = jax.lax.rsqrt(jnp.maximum(jnp.sum(blk, axis=1, keepdims=True), 1.0))
    dis_ref[...] = dis
    z = jnp.dot(x_ref[...], w0_ref[...], preferred_element_type=jnp.float32)
    z1_ref[...] = (dis * z).astype(jnp.bfloat16)


def _layers_body(br, z1_ref, dis_ref, bs_ref, ws_ref, abf_ref, out_ref,
                 z2_ref, z3_ref):
    l = pl.program_id(0)
    i = pl.program_id(1)
    a = abf_ref[...]
    dis = dis_ref[...]
    b = bs_ref[0]
    w = ws_ref[0]

    def step(z_in, z_out):
        acc = jnp.dot(a, z_in, preferred_element_type=jnp.float32)
        h = jnp.maximum(acc * dis + b, 0.0)
        if z_out is None:
            out_ref[...] = h
        else:
            z = jnp.dot(h, w, preferred_element_type=jnp.float32)
            z_out[pl.ds(i * br, br), :] = (dis * z).astype(jnp.bfloat16)

    @pl.when(l == 0)
    def _():
        step(z1_ref[...], z2_ref)

    @pl.when(l == 1)
    def _():
        step(z2_ref[...], z3_ref)

    @pl.when(l == 2)
    def _():
        step(z3_ref[...], None)


def _prep(adj, x, w0):
    n, f = x.shape
    return pl.pallas_call(
        functools.partial(_prep_body, _BR_P),
        grid=(n // _BR_P,),
        in_specs=[
            pl.BlockSpec((_BR_P, n), lambda i: (i, 0)),
            pl.BlockSpec((_BR_P, f), lambda i: (i, 0)),
            pl.BlockSpec((f, f), lambda i: (0, 0)),
        ],
        out_specs=[
            pl.BlockSpec((_BR_P, n), lambda i: (i, 0)),
            pl.BlockSpec((_BR_P, 1), lambda i: (i, 0)),
            pl.BlockSpec((_BR_P, f), lambda i: (i, 0)),
        ],
        out_shape=[
            jax.ShapeDtypeStruct((n, n), jnp.bfloat16),
            jax.ShapeDtypeStruct((n, 1), jnp.float32),
            jax.ShapeDtypeStruct((n, f), jnp.bfloat16),
        ],
        compiler_params=pltpu.CompilerParams(
            dimension_semantics=("arbitrary",)),
    )(adj, x, w0)


def _layers(z1, dis, b_stack, w_stack, abf):
    n = abf.shape[0]
    f = z1.shape[1]
    return pl.pallas_call(
        functools.partial(_layers_body, _BR),
        grid=(3, n // _BR),
        in_specs=[
            pl.BlockSpec((n, f), lambda l, i: (0, 0)),
            pl.BlockSpec((_BR, 1), lambda l, i: (i, 0)),
            pl.BlockSpec((1, 1, f), lambda l, i: (l, 0, 0)),
            pl.BlockSpec((1, f, f), lambda l, i: (l, 0, 0)),
            pl.BlockSpec((_BR, n), lambda l, i: (i, 0)),
        ],
        out_specs=pl.BlockSpec((_BR, f),
                               lambda l, i: (jnp.where(l == 2, i, 0), 0)),
        out_shape=jax.ShapeDtypeStruct((n, f), jnp.float32),
        scratch_shapes=[
            pltpu.VMEM((n, f), jnp.bfloat16),
            pltpu.VMEM((n, f), jnp.bfloat16),
        ],
        compiler_params=pltpu.CompilerParams(
            dimension_semantics=("arbitrary", "arbitrary")),
    )(z1, dis, b_stack, w_stack, abf)


def kernel(x, adj, W0, b0, W1, b1, W2, b2):
    abf, dis, z1 = _prep(adj, x, W0)
    b_stack = jnp.stack([b0, b1, b2]).reshape(3, 1, -1)
    w_stack = jnp.stack([W1, W2, W2])
    return _layers(z1, dis, b_stack, w_stack, abf)


# speedup vs baseline: 1.1141x; 1.1141x over previous
import functools

import jax
import jax.numpy as jnp
from jax.experimental import pallas as pl
from jax.experimental.pallas import tpu as pltpu

_BR_P = 200
_BR = 1000


def _prep_body(br, adj_ref, x_ref, w0_ref, abf_ref, dis_ref, z1_ref):
    i = pl.program_id(0)
    blk = adj_ref[...]
    rows = jax.lax.broadcasted_iota(jnp.int32, blk.shape, 0) + i * br
    cols = jax.lax.broadcasted_iota(jnp.int32, blk.shape, 1)
    blk = jnp.where(rows == cols, 1.0, blk)
    abf_ref[...] = blk.astype(jnp.bfloat16)
    dis = jax.lax.rsqrt(jnp.maximum(jnp.sum(blk, axis=1, keepdims=True), 1.0))
    dis_ref[...] = dis
    z = jnp.dot(x_ref[...], w0_ref[...], preferred_element_type=jnp.float32)
    z1_ref[...] = (dis * z).astype(jnp.bfloat16)


def _layers_body(br, z1_ref, dis_ref, bs_ref, ws_ref, abf_ref, out_ref,
                 z2_ref, z3_ref):
    l = pl.program_id(0)
    i = pl.program_id(1)
    dis = dis_ref[...]
    b = bs_ref[0]
    w = ws_ref[0]

    def step(z_in, z_out):
        acc = jnp.dot(abf_ref[...], z_in, preferred_element_type=jnp.float32)
        h = jnp.maximum(acc * dis + b, 0.0)
        if z_out is None:
            out_ref[...] = h
        else:
            z = jnp.dot(h, w, preferred_element_type=jnp.float32)
            z_out[pl.ds(i * br, br), :] = (dis * z).astype(jnp.bfloat16)

    @pl.when(l == 0)
    def _():
        step(z1_ref[...], z2_ref)

    @pl.when(l == 1)
    def _():
        step(z2_ref[...], z3_ref)

    @pl.when(l == 2)
    def _():
        step(z3_ref[...], None)


def _prep(adj, x, w0):
    n, f = x.shape
    return pl.pallas_call(
        functools.partial(_prep_body, _BR_P),
        grid=(n // _BR_P,),
        in_specs=[
            pl.BlockSpec((_BR_P, n), lambda i: (i, 0)),
            pl.BlockSpec((_BR_P, f), lambda i: (i, 0)),
            pl.BlockSpec((f, f), lambda i: (0, 0)),
        ],
        out_specs=[
            pl.BlockSpec((_BR_P, n), lambda i: (i, 0)),
            pl.BlockSpec((_BR_P, 1), lambda i: (i, 0)),
            pl.BlockSpec((_BR_P, f), lambda i: (i, 0)),
        ],
        out_shape=[
            jax.ShapeDtypeStruct((n, n), jnp.bfloat16),
            jax.ShapeDtypeStruct((n, 1), jnp.float32),
            jax.ShapeDtypeStruct((n, f), jnp.bfloat16),
        ],
        compiler_params=pltpu.CompilerParams(
            dimension_semantics=("arbitrary",)),
    )(adj, x, w0)


def _layers(z1, dis, b_stack, w_stack, abf):
    n = abf.shape[0]
    f = z1.shape[1]
    return pl.pallas_call(
        functools.partial(_layers_body, _BR),
        grid=(3, n // _BR),
        in_specs=[
            pl.BlockSpec((n, f), lambda l, i: (0, 0)),
            pl.BlockSpec((_BR, 1), lambda l, i: (i, 0)),
            pl.BlockSpec((1, 1, f), lambda l, i: (l, 0, 0)),
            pl.BlockSpec((1, f, f), lambda l, i: (l, 0, 0)),
            pl.BlockSpec((_BR, n), lambda l, i: (i, 0)),
        ],
        out_specs=pl.BlockSpec((_BR, f),
                               lambda l, i: (jnp.where(l == 2, i, 0), 0)),
        out_shape=jax.ShapeDtypeStruct((n, f), jnp.float32),
        scratch_shapes=[
            pltpu.VMEM((n, f), jnp.bfloat16),
            pltpu.VMEM((n, f), jnp.bfloat16),
        ],
        compiler_params=pltpu.CompilerParams(
            dimension_semantics=("arbitrary", "arbitrary")),
    )(z1, dis, b_stack, w_stack, abf)


def kernel(x, adj, W0, b0, W1, b1, W2, b2):
    abf, dis, z1 = _prep(adj, x, W0)
    b_stack = jnp.stack([b0, b1, b2]).reshape(3, 1, -1)
    w_stack = jnp.stack([W1, W2, W2])
    return _layers(z1, dis, b_stack, w_stack, abf)


# speedup vs baseline: 1.1234x; 1.0083x over previous
import functools

import jax
import jax.numpy as jnp
from jax.experimental import pallas as pl
from jax.experimental.pallas import tpu as pltpu

_BR_P = 400
_BR = 1000


def _prep_body(br, adj_ref, x_ref, w0_ref, abf_ref, dis_ref, z1_ref):
    i = pl.program_id(0)
    blk = adj_ref[...]
    rows = jax.lax.broadcasted_iota(jnp.int32, blk.shape, 0) + i * br
    cols = jax.lax.broadcasted_iota(jnp.int32, blk.shape, 1)
    blk = jnp.where(rows == cols, 1.0, blk)
    abf_ref[...] = blk.astype(jnp.bfloat16)
    dis = jax.lax.rsqrt(jnp.maximum(jnp.sum(blk, axis=1, keepdims=True), 1.0))
    dis_ref[...] = dis
    z = jnp.dot(x_ref[...], w0_ref[...], preferred_element_type=jnp.float32)
    z1_ref[...] = (dis * z).astype(jnp.bfloat16)


def _layers_body(br, z1_ref, dis_ref, bs_ref, ws_ref, abf_ref, out_ref,
                 z2_ref, z3_ref):
    l = pl.program_id(0)
    i = pl.program_id(1)
    dis = dis_ref[...]
    b = bs_ref[0]
    w = ws_ref[0]

    def step(z_in, z_out):
        acc = jnp.dot(abf_ref[...], z_in, preferred_element_type=jnp.float32)
        h = jnp.maximum(acc * dis + b, 0.0)
        if z_out is None:
            out_ref[...] = h
        else:
            z = jnp.dot(h, w, preferred_element_type=jnp.float32)
            z_out[pl.ds(i * br, br), :] = (dis * z).astype(jnp.bfloat16)

    @pl.when(l == 0)
    def _():
        step(z1_ref[...], z2_ref)

    @pl.when(l == 1)
    def _():
        step(z2_ref[...], z3_ref)

    @pl.when(l == 2)
    def _():
        step(z3_ref[...], None)


def _prep(adj, x, w0):
    n, f = x.shape
    return pl.pallas_call(
        functools.partial(_prep_body, _BR_P),
        grid=(n // _BR_P,),
        in_specs=[
            pl.BlockSpec((_BR_P, n), lambda i: (i, 0)),
            pl.BlockSpec((_BR_P, f), lambda i: (i, 0)),
            pl.BlockSpec((f, f), lambda i: (0, 0)),
        ],
        out_specs=[
            pl.BlockSpec((_BR_P, n), lambda i: (i, 0)),
            pl.BlockSpec((_BR_P, 1), lambda i: (i, 0)),
            pl.BlockSpec((_BR_P, f), lambda i: (i, 0)),
        ],
        out_shape=[
            jax.ShapeDtypeStruct((n, n), jnp.bfloat16),
            jax.ShapeDtypeStruct((n, 1), jnp.float32),
            jax.ShapeDtypeStruct((n, f), jnp.bfloat16),
        ],
        compiler_params=pltpu.CompilerParams(
            dimension_semantics=("parallel",),
            vmem_limit_bytes=64 * 1024 * 1024),
    )(adj, x, w0)


def _layers(z1, dis, b_stack, w_stack, abf):
    n = abf.shape[0]
    f = z1.shape[1]
    return pl.pallas_call(
        functools.partial(_layers_body, _BR),
        grid=(3, n // _BR),
        in_specs=[
            pl.BlockSpec((n, f), lambda l, i: (0, 0)),
            pl.BlockSpec((_BR, 1), lambda l, i: (i, 0)),
            pl.BlockSpec((1, 1, f), lambda l, i: (l, 0, 0)),
            pl.BlockSpec((1, f, f), lambda l, i: (l, 0, 0)),
            pl.BlockSpec((_BR, n), lambda l, i: (i, 0)),
        ],
        out_specs=pl.BlockSpec((_BR, f),
                               lambda l, i: (jnp.where(l == 2, i, 0), 0)),
        out_shape=jax.ShapeDtypeStruct((n, f), jnp.float32),
        scratch_shapes=[
            pltpu.VMEM((n, f), jnp.bfloat16),
            pltpu.VMEM((n, f), jnp.bfloat16),
        ],
        compiler_params=pltpu.CompilerParams(
            dimension_semantics=("arbitrary", "arbitrary")),
    )(z1, dis, b_stack, w_stack, abf)


def kernel(x, adj, W0, b0, W1, b1, W2, b2):
    abf, dis, z1 = _prep(adj, x, W0)
    b_stack = jnp.stack([b0, b1, b2]).reshape(3, 1, -1)
    w_stack = jnp.stack([W1, W2, W2])
    return _layers(z1, dis, b_stack, w_stack, abf)
